# Initial kernel scaffold; baseline (speedup 1.0000x reference)
import jax
import jax.numpy as jnp
from jax.experimental import pallas as pl


def kernel(x, distance_sum, scale):
    raise NotImplementedError("write your pallas kernel here")


# speedup vs baseline: 3.0841x; 3.0841x over previous
import functools

import jax
import jax.numpy as jnp
import numpy as np
from jax.experimental import pallas as pl
from jax.experimental.pallas import tpu as pltpu

_BINS = np.asarray(
    [-1, 10, 20, 30, 40, 50, 60, 70, 80, 90, 100, 110, 120, 130, 140, 150,
     170, 190, 224],
    dtype=np.float32,
)
_NUM_BINS = _BINS.shape[0] - 1
_PAD = 128

_BLK = 1024


def _body(d_ref, x_ref, lo_ref, hi_ref, scale_ref, o_ref):
    d = d_ref[...]
    lo = lo_ref[...]
    hi = hi_ref[...]
    onehot = jnp.logical_and(d > lo, d <= hi).astype(jnp.float32)
    gathered = jnp.dot(onehot, scale_ref[...],
                       preferred_element_type=jnp.float32)
    o_ref[...] = x_ref[...] * jnp.exp(-gathered * d)


@jax.jit
def kernel(x, distance_sum, scale):
    n, f = x.shape
    d2 = distance_sum.reshape(n, 1)
    lo = np.full((1, _PAD), np.inf, np.float32)
    hi = np.full((1, _PAD), np.inf, np.float32)
    lo[0, :_NUM_BINS] = _BINS[:-1]
    hi[0, :_NUM_BINS] = _BINS[1:]
    lo = jnp.asarray(lo)
    hi = jnp.asarray(hi)
    scale_pad = jnp.zeros((_PAD, f), jnp.float32).at[:_NUM_BINS].set(scale)

    grid = (n // _BLK,)
    return pl.pallas_call(
        _body,
        grid=grid,
        in_specs=[
            pl.BlockSpec((_BLK, 1), lambda i: (i, 0)),
            pl.BlockSpec((_BLK, f), lambda i: (i, 0)),
            pl.BlockSpec((1, _PAD), lambda i: (0, 0)),
            pl.BlockSpec((1, _PAD), lambda i: (0, 0)),
            pl.BlockSpec((_PAD, f), lambda i: (0, 0)),
        ],
        out_specs=pl.BlockSpec((_BLK, f), lambda i: (i, 0)),
        out_shape=jax.ShapeDtypeStruct((n, f), jnp.float32),
        compiler_params=pltpu.CompilerParams(
            dimension_semantics=("arbitrary",),
        ),
    )(d2, x, lo, hi, scale_pad)
